# Initial kernel scaffold; baseline (speedup 1.0000x reference)
import jax
import jax.numpy as jnp
from jax.experimental import pallas as pl


def kernel(x, pos, edge_index, edge_attr, batch, W_in, b_in, msg_W1, msg_b1, msg_W2, msg_b2, upd_W1, upd_b1, upd_W2, upd_b2, head_e_W, head_e_b, head_i_W, head_i_b):
    raise NotImplementedError("write your pallas kernel here")


# speedup vs baseline: 1.8924x; 1.8924x over previous
import functools

import jax
import jax.numpy as jnp
from jax import lax
from jax.experimental import pallas as pl
from jax.experimental.pallas import tpu as pltpu
import jax.experimental.pallas.tpu_sc as plsc

N = 50000
E = 800000
D = 64
G = 256
NP = 51200
EP = 819200
MROWS = EP // 128
NBLK = NP // 512
EBLK = EP // 1024
TSL = NP // 16
WR = EP // 128 // 32
SROW = EP // 16 // 128
NSUP = SROW // 8
PASSES = ((0, 104), (104, 96))

_mesh = plsc.VectorSubcoreMesh(core_axis_name="c", subcore_axis_name="s")
_f32 = jnp.float32


def _prep_body(pos16, dst2, src2, pi_out, pj_out, deg_out,
               idx_d, idx_s, bufPi, bufPj, obuf, zbuf, degsp, sem):
    c = lax.axis_index("c")
    s = lax.axis_index("s")
    w = c * 16 + s

    @pl.loop(0, TSL // 16)
    def _z(i):
        zbuf[pl.ds(i * 16, 16)] = jnp.zeros((16,), _f32)

    pltpu.sync_copy(zbuf, degsp.at[pl.ds(s * TSL, TSL)])
    for i in range(8):
        obuf[pl.ds(i * 16, 16)] = jnp.ones((16,), _f32)
    plsc.subcore_barrier()

    for prow, pn in PASSES:
        row0 = w * WR + prow
        pltpu.sync_copy(dst2.at[pl.ds(row0, pn)], idx_d.at[pl.ds(0, pn)])
        pltpu.sync_copy(src2.at[pl.ds(row0, pn)], idx_s.at[pl.ds(0, pn)])

        @pl.loop(0, pn // 8)
        def _g(g):
            descs = []
            for j in range(8):
                descs.append(pltpu.async_copy(
                    pos16.at[idx_d.at[g * 8 + j]],
                    bufPi.at[pl.ds(j * 128, 128)], sem))
                descs.append(pltpu.async_copy(
                    pos16.at[idx_s.at[g * 8 + j]],
                    bufPj.at[pl.ds(j * 128, 128)], sem))
            for dd in descs:
                dd.wait()
            e0 = (row0 + g * 8) * 128
            pltpu.sync_copy(bufPi, pi_out.at[pl.ds(e0, 1024)])
            pltpu.sync_copy(bufPj, pj_out.at[pl.ds(e0, 1024)])
            for j in range(8):
                pltpu.sync_copy(obuf, degsp.at[idx_d.at[g * 8 + j]], add=True)

    plsc.subcore_barrier()
    pltpu.sync_copy(degsp.at[pl.ds(s * TSL, TSL)],
                    deg_out.at[pl.ds(c * NP + s * TSL, TSL)])


_prep = pl.kernel(
    _prep_body,
    out_type=[jax.ShapeDtypeStruct((EP, 16), _f32),
              jax.ShapeDtypeStruct((EP, 16), _f32),
              jax.ShapeDtypeStruct((2 * NP,), _f32)],
    mesh=_mesh,
    scratch_types=[pltpu.VMEM((104, 128), jnp.int32),
                   pltpu.VMEM((104, 128), jnp.int32),
                   pltpu.VMEM((1024, 16), _f32),
                   pltpu.VMEM((1024, 16), _f32),
                   pltpu.VMEM((128,), _f32),
                   pltpu.VMEM((TSL,), _f32),
                   pltpu.VMEM_SHARED((NP,), _f32),
                   pltpu.SemaphoreType.DMA],
    compiler_params=pltpu.CompilerParams(use_tc_tiling_on_sc=False, needs_layout_passes=False),
)


def _msg_body(a_tab, b_tab, c_all, dst2, src2, lsel, m_out,
              idx_d, idx_s, bufA, bufB, bufC, bufP, lbuf, sem):
    c = lax.axis_index("c")
    s = lax.axis_index("s")
    coff = c * NP
    pltpu.sync_copy(lsel, lbuf)
    lv = jnp.sum(lbuf[...])

    @pl.loop(0, NSUP)
    def _g(g):
        row0 = s * SROW + g * 8
        e0 = row0 * 128
        pltpu.sync_copy(dst2.at[pl.ds(row0, 8)], idx_d)
        pltpu.sync_copy(src2.at[pl.ds(row0, 8)], idx_s)

        @pl.loop(0, 8)
        def _t(j):
            for h in range(8):
                idx_d[j, pl.ds(h * 16, 16)] = idx_d[j, pl.ds(h * 16, 16)] + coff
                idx_s[j, pl.ds(h * 16, 16)] = idx_s[j, pl.ds(h * 16, 16)] + coff

        cpd = pltpu.async_copy(c_all.at[c, lv, pl.ds(e0, 1024)], bufC, sem)
        descs = []
        for j in range(8):
            descs.append(pltpu.async_copy(
                a_tab.at[idx_d.at[j]], bufA.at[pl.ds(j * 128, 128)], sem))
            descs.append(pltpu.async_copy(
                b_tab.at[idx_s.at[j]], bufB.at[pl.ds(j * 128, 128)], sem))
        cpd.wait()
        for dsc in descs:
            dsc.wait()

        @pl.loop(0, 128)
        def _r(rr):
            base = rr * 8
            for u in range(8):
                r = base + u
                v0 = (bufA[r, pl.ds(0, 16)] + bufB[r, pl.ds(0, 16)]
                      + bufC[r, pl.ds(0, 16)])
                v1 = (bufA[r, pl.ds(16, 16)] + bufB[r, pl.ds(16, 16)]
                      + bufC[r, pl.ds(16, 16)])
                bufP[r, pl.ds(0, 32)] = plsc.pack(
                    jnp.maximum(v0, 0.0), jnp.maximum(v1, 0.0),
                    format=plsc.PackFormat.INTERLEAVED)

        pltpu.sync_copy(bufP, m_out.at[c, pl.ds(e0, 1024)])


_msg = pl.kernel(
    _msg_body,
    out_type=[jax.ShapeDtypeStruct((2, EP, 32), jnp.bfloat16)],
    mesh=_mesh,
    scratch_types=[pltpu.VMEM((8, 128), jnp.int32),
                   pltpu.VMEM((8, 128), jnp.int32),
                   pltpu.VMEM((1024, 32), _f32),
                   pltpu.VMEM((1024, 32), _f32),
                   pltpu.VMEM((1024, 32), _f32),
                   pltpu.VMEM((1024, 32), jnp.bfloat16),
                   pltpu.VMEM((16,), jnp.int32),
                   pltpu.SemaphoreType.DMA],
    compiler_params=pltpu.CompilerParams(use_tc_tiling_on_sc=False, needs_layout_passes=False),
)


NP2 = NP // 2
ACC_R = NP2 + 128
DSL = NP2 // 16


def _agg_body(m, dst2, s_out, idx_d, idxl, bufM, accsp, sem):
    c = lax.axis_index("c")
    s = lax.axis_index("s")
    for half in range(2):
        off = half * NP2

        @pl.loop(0, 800)
        def _z(r):
            bufM[r, pl.ds(0, 32)] = jnp.zeros((32,), jnp.bfloat16)

        for q in range(2):
            pltpu.sync_copy(bufM.at[pl.ds(0, 800)],
                            accsp.at[pl.ds(s * DSL + q * 800, 800)])
        plsc.subcore_barrier()

        @pl.loop(0, NSUP)
        def _g(g):
            row0 = s * SROW + g * 8
            e0 = row0 * 128
            pltpu.sync_copy(dst2.at[pl.ds(row0, 8)], idx_d)
            cpm = pltpu.async_copy(m.at[c, pl.ds(e0, 1024)], bufM, sem)

            @pl.loop(0, 8)
            def _t(j):
                for h in range(8):
                    loc = idx_d[j, pl.ds(h * 16, 16)] - off
                    bad = (loc < 0) | (loc >= NP2)
                    idxl[j, pl.ds(h * 16, 16)] = jnp.where(bad, NP2, loc)

            cpm.wait()
            for j in range(8):
                pltpu.sync_copy(bufM.at[pl.ds(j * 128, 128)],
                                accsp.at[idxl.at[j]], add=True)

        plsc.subcore_barrier()
        pltpu.sync_copy(accsp.at[pl.ds(s * DSL, DSL)],
                        s_out.at[c, pl.ds(off + s * DSL, DSL)])


_agg = pl.kernel(
    _agg_body,
    out_type=[jax.ShapeDtypeStruct((2, NP, 32), jnp.bfloat16)],
    mesh=_mesh,
    scratch_types=[pltpu.VMEM((8, 128), jnp.int32),
                   pltpu.VMEM((8, 128), jnp.int32),
                   pltpu.VMEM((1024, 32), jnp.bfloat16),
                   pltpu.VMEM_SHARED((ACC_R, 32), jnp.bfloat16),
                   pltpu.SemaphoreType.DMA],
    compiler_params=pltpu.CompilerParams(use_tc_tiling_on_sc=False, needs_layout_passes=False),
)


def _lin_in_body(x_ref, w_ref, b_ref, o_ref):
    o_ref[...] = jnp.dot(x_ref[...], w_ref[...],
                         preferred_element_type=_f32) + b_ref[...]


def _ab_body(h_ref, wi_ref, wj_ref, a_ref, b_ref):
    i = pl.program_id(0)
    h = h_ref[...]
    A = jnp.dot(h, wi_ref[...], preferred_element_type=_f32)
    B = jnp.dot(h, wj_ref[...], preferred_element_type=_f32)
    lo = i < NBLK
    a_ref[...] = jnp.where(lo, A[:, :32], A[:, 32:])
    b_ref[...] = jnp.where(lo, B[:, :32], B[:, 32:])


def _c_body(ea_ref, pi_ref, pj_ref, w_ref, wd_ref, b_ref, c_ref):
    dp = pi_ref[...] - pj_ref[...]
    d2 = jnp.sum(dp * dp, axis=1, keepdims=True)
    R = jnp.dot(ea_ref[...], w_ref[...], preferred_element_type=_f32)
    R = R + d2 * wd_ref[...] + b_ref[...]
    for cc in range(2):
        for l in range(4):
            c_ref[cc, l] = R[:, l * 64 + cc * 32: l * 64 + (cc + 1) * 32]


def _upd_body(h_ref, s2_ref, deg_ref, w2_ref, b2_ref,
              u1h_ref, u1a_ref, ub1_ref, u2_ref, ub2_ref, o_ref):
    S = jnp.concatenate([s2_ref[0], s2_ref[1]], axis=1).astype(_f32)
    degs = deg_ref[0] + deg_ref[1]
    agg = jnp.dot(S, w2_ref[...], preferred_element_type=_f32) \
        + degs * b2_ref[...]
    t = jnp.maximum(
        jnp.dot(h_ref[...], u1h_ref[...], preferred_element_type=_f32)
        + jnp.dot(agg, u1a_ref[...], preferred_element_type=_f32)
        + ub1_ref[...], 0.0)
    o_ref[...] = jnp.dot(t, u2_ref[...], preferred_element_type=_f32) \
        + ub2_ref[...]


def _pool_body(h_ref, b3_ref, hw_ref, hb_ref, o_ref, acc, cnt):
    i = pl.program_id(0)

    @pl.when(i == 0)
    def _():
        acc[...] = jnp.zeros_like(acc)
        cnt[...] = jnp.zeros_like(cnt)

    bv = b3_ref[0]
    iota = lax.broadcasted_iota(jnp.int32, (1, G), 1)
    oh = (bv == iota).astype(_f32)
    acc[...] += lax.dot_general(oh, h_ref[...], (((0,), (0,)), ((), ())),
                                preferred_element_type=_f32)
    cnt[...] += lax.dot_general(oh, jnp.ones((512, 1), _f32),
                                (((0,), (0,)), ((), ())),
                                preferred_element_type=_f32)

    @pl.when(i == NBLK - 1)
    def _():
        pooled = acc[...] / jnp.maximum(cnt[...], 1.0)
        o_ref[...] = jnp.dot(pooled, hw_ref[...],
                             preferred_element_type=_f32) + hb_ref[...]


_lin_in = pl.pallas_call(
    _lin_in_body,
    grid=(NBLK,),
    in_specs=[pl.BlockSpec((512, 16), lambda i: (i, 0)),
              pl.BlockSpec((16, 64), lambda i: (0, 0)),
              pl.BlockSpec((1, 64), lambda i: (0, 0))],
    out_specs=pl.BlockSpec((512, 64), lambda i: (i, 0)),
    out_shape=jax.ShapeDtypeStruct((NP, 64), _f32),
)

_ab = pl.pallas_call(
    _ab_body,
    grid=(2 * NBLK,),
    in_specs=[pl.BlockSpec((512, 64), lambda i: (i % NBLK, 0)),
              pl.BlockSpec((64, 64), lambda i: (0, 0)),
              pl.BlockSpec((64, 64), lambda i: (0, 0))],
    out_specs=[pl.BlockSpec((512, 32), lambda i: (i, 0)),
               pl.BlockSpec((512, 32), lambda i: (i, 0))],
    out_shape=[jax.ShapeDtypeStruct((2 * NP, 32), _f32),
               jax.ShapeDtypeStruct((2 * NP, 32), _f32)],
)

_cmat = pl.pallas_call(
    _c_body,
    grid=(EBLK,),
    in_specs=[pl.BlockSpec((1024, 16), lambda i: (i, 0)),
              pl.BlockSpec((1024, 16), lambda i: (i, 0)),
              pl.BlockSpec((1024, 16), lambda i: (i, 0)),
              pl.BlockSpec((16, 256), lambda i: (0, 0)),
              pl.BlockSpec((1, 256), lambda i: (0, 0)),
              pl.BlockSpec((1, 256), lambda i: (0, 0))],
    out_specs=pl.BlockSpec((2, 4, 1024, 32), lambda i: (0, 0, i, 0)),
    out_shape=jax.ShapeDtypeStruct((2, 4, EP, 32), _f32),
)

_upd = pl.pallas_call(
    _upd_body,
    grid=(NBLK,),
    in_specs=[pl.BlockSpec((512, 64), lambda i: (i, 0)),
              pl.BlockSpec((2, 512, 32), lambda i: (0, i, 0)),
              pl.BlockSpec((2, 512, 1), lambda i: (0, i, 0)),
              pl.BlockSpec((64, 64), lambda i: (0, 0)),
              pl.BlockSpec((1, 64), lambda i: (0, 0)),
              pl.BlockSpec((64, 64), lambda i: (0, 0)),
              pl.BlockSpec((64, 64), lambda i: (0, 0)),
              pl.BlockSpec((1, 64), lambda i: (0, 0)),
              pl.BlockSpec((64, 64), lambda i: (0, 0)),
              pl.BlockSpec((1, 64), lambda i: (0, 0))],
    out_specs=pl.BlockSpec((512, 64), lambda i: (i, 0)),
    out_shape=jax.ShapeDtypeStruct((NP, 64), _f32),
)

_pool = pl.pallas_call(
    _pool_body,
    grid=(NBLK,),
    in_specs=[pl.BlockSpec((512, 64), lambda i: (i, 0)),
              pl.BlockSpec((1, 512, 1), lambda i: (i, 0, 0)),
              pl.BlockSpec((64, 600), lambda i: (0, 0)),
              pl.BlockSpec((1, 600), lambda i: (0, 0))],
    out_specs=pl.BlockSpec((G, 600), lambda i: (0, 0)),
    out_shape=jax.ShapeDtypeStruct((G, 600), _f32),
    scratch_shapes=[pltpu.VMEM((G, 64), _f32), pltpu.VMEM((G, 1), _f32)],
    compiler_params=pltpu.CompilerParams(
        dimension_semantics=("arbitrary",)),
)


def kernel(x, pos, edge_index, edge_attr, batch, W_in, b_in,
           msg_W1, msg_b1, msg_W2, msg_b2,
           upd_W1, upd_b1, upd_W2, upd_b2,
           head_e_W, head_e_b, head_i_W, head_i_b):
    f32 = _f32
    xp = jnp.pad(x, ((0, NP - N), (0, 5)))
    pos16 = jnp.pad(pos, ((0, NP - N), (0, 13)))
    dst2 = jnp.concatenate(
        [edge_index[1], jnp.full((EP - E,), N, jnp.int32)]).reshape(MROWS, 128)
    src2 = jnp.concatenate(
        [edge_index[0], jnp.full((EP - E,), N, jnp.int32)]).reshape(MROWS, 128)
    eap = jnp.pad(edge_attr, ((0, EP - E), (0, 0)))
    W_inp = jnp.pad(W_in, ((0, 5), (0, 0)))
    W1e_cat = jnp.concatenate([msg_W1[l, 128:144] for l in range(4)], axis=1)
    w1d_cat = jnp.concatenate([msg_W1[l, 144:145] for l in range(4)], axis=1)
    b1_cat = msg_b1.reshape(1, 256)
    head_W = jnp.concatenate([head_e_W, head_i_W], axis=1)
    head_b = jnp.concatenate([head_e_b, head_i_b]).reshape(1, 600)
    batch3 = jnp.pad(batch, (0, NP - N),
                     constant_values=G).reshape(NBLK, 512, 1)
    import numpy as _np
    j32 = _np.arange(32)
    p32 = (j32 % 2) * 16 + j32 // 2
    permn = _np.concatenate([p32, p32 + 32])
    Pm = jnp.asarray(_np.eye(64, dtype=_np.float32)[permn])
    W2p_all = jnp.einsum("kj,ljm->lkm", Pm, msg_W2)

    pos_i, pos_j, deg2 = _prep(pos16, dst2, src2)
    deg3 = deg2.reshape(2, NP, 1)

    c_all = _cmat(eap, pos_i, pos_j, W1e_cat, w1d_cat, b1_cat)

    h = _lin_in(xp, W_inp, b_in.reshape(1, 64))
    lsel_all = jnp.concatenate(
        [jnp.arange(4, dtype=jnp.int32).reshape(4, 1),
         jnp.zeros((4, 15), jnp.int32)], axis=1)
    xs = (msg_W1[:, :64], msg_W1[:, 64:128], W2p_all,
          msg_b2.reshape(4, 1, 64), upd_W1[:, :64], upd_W1[:, 64:],
          upd_b1.reshape(4, 1, 64), upd_W2, upd_b2.reshape(4, 1, 64),
          lsel_all)

    def _layer(hc, x):
        w1i, w1j, w2p, b2, u1h, u1a, ub1, u2, ub2, lsel = x
        a_tab, b_tab = _ab(hc, w1i, w1j)
        (m,) = _msg(a_tab, b_tab, c_all, dst2, src2, lsel)
        (s2,) = _agg(m, dst2)
        hn = _upd(hc, s2, deg3, w2p, b2, u1h, u1a, ub1, u2, ub2)
        return hn, None

    h, _ = lax.scan(_layer, h, xs)

    return _pool(h, batch3, head_W, head_b)


# speedup vs baseline: 1.9207x; 1.0150x over previous
import functools

import jax
import jax.numpy as jnp
from jax import lax
from jax.experimental import pallas as pl
from jax.experimental.pallas import tpu as pltpu
import jax.experimental.pallas.tpu_sc as plsc

N = 50000
E = 800000
D = 64
G = 256
NP = 51200
EP = 819200
MROWS = EP // 128
NBLK = NP // 512
EBLK = EP // 1024
TSL = NP // 16
WR = EP // 128 // 32
SROW = EP // 16 // 128
NSUP = SROW // 8
PASSES = ((0, 104), (104, 96))

_mesh = plsc.VectorSubcoreMesh(core_axis_name="c", subcore_axis_name="s")
_f32 = jnp.float32


def _prep_body(pos16, dst2, src2, pi_out, pj_out, deg_out,
               idx_d, idx_s, bufPi, bufPj, obuf, zbuf, degsp, sem):
    c = lax.axis_index("c")
    s = lax.axis_index("s")
    w = c * 16 + s

    @pl.loop(0, TSL // 16)
    def _z(i):
        zbuf[pl.ds(i * 16, 16)] = jnp.zeros((16,), _f32)

    pltpu.sync_copy(zbuf, degsp.at[pl.ds(s * TSL, TSL)])
    for i in range(8):
        obuf[pl.ds(i * 16, 16)] = jnp.ones((16,), _f32)
    plsc.subcore_barrier()

    for prow, pn in PASSES:
        row0 = w * WR + prow
        pltpu.sync_copy(dst2.at[pl.ds(row0, pn)], idx_d.at[pl.ds(0, pn)])
        pltpu.sync_copy(src2.at[pl.ds(row0, pn)], idx_s.at[pl.ds(0, pn)])

        @pl.loop(0, pn // 8)
        def _g(g):
            descs = []
            for j in range(8):
                descs.append(pltpu.async_copy(
                    pos16.at[idx_d.at[g * 8 + j]],
                    bufPi.at[pl.ds(j * 128, 128)], sem))
                descs.append(pltpu.async_copy(
                    pos16.at[idx_s.at[g * 8 + j]],
                    bufPj.at[pl.ds(j * 128, 128)], sem))
            for dd in descs:
                dd.wait()
            e0 = (row0 + g * 8) * 128
            pltpu.sync_copy(bufPi, pi_out.at[pl.ds(e0, 1024)])
            pltpu.sync_copy(bufPj, pj_out.at[pl.ds(e0, 1024)])
            for j in range(8):
                pltpu.sync_copy(obuf, degsp.at[idx_d.at[g * 8 + j]], add=True)

    plsc.subcore_barrier()
    pltpu.sync_copy(degsp.at[pl.ds(s * TSL, TSL)],
                    deg_out.at[pl.ds(c * NP + s * TSL, TSL)])


_prep = pl.kernel(
    _prep_body,
    out_type=[jax.ShapeDtypeStruct((EP, 16), _f32),
              jax.ShapeDtypeStruct((EP, 16), _f32),
              jax.ShapeDtypeStruct((2 * NP,), _f32)],
    mesh=_mesh,
    scratch_types=[pltpu.VMEM((104, 128), jnp.int32),
                   pltpu.VMEM((104, 128), jnp.int32),
                   pltpu.VMEM((1024, 16), _f32),
                   pltpu.VMEM((1024, 16), _f32),
                   pltpu.VMEM((128,), _f32),
                   pltpu.VMEM((TSL,), _f32),
                   pltpu.VMEM_SHARED((NP,), _f32),
                   pltpu.SemaphoreType.DMA],
    compiler_params=pltpu.CompilerParams(use_tc_tiling_on_sc=False, needs_layout_passes=False),
)


def _msg_body(a_tab, b_tab, c_all, dst2, src2, lsel, m_out,
              idx_d, idx_s, bufA, bufB, bufC, bufP, lbuf, sem):
    c = lax.axis_index("c")
    s = lax.axis_index("s")
    coff = c * NP
    pltpu.sync_copy(lsel, lbuf)
    lv = jnp.sum(lbuf[...])

    @pl.loop(0, NSUP)
    def _g(g):
        row0 = s * SROW + g * 8
        e0 = row0 * 128
        pltpu.sync_copy(dst2.at[pl.ds(row0, 8)], idx_d)
        pltpu.sync_copy(src2.at[pl.ds(row0, 8)], idx_s)

        @pl.loop(0, 8)
        def _t(j):
            for h in range(8):
                idx_d[j, pl.ds(h * 16, 16)] = idx_d[j, pl.ds(h * 16, 16)] + coff
                idx_s[j, pl.ds(h * 16, 16)] = idx_s[j, pl.ds(h * 16, 16)] + coff

        cpd = pltpu.async_copy(c_all.at[c, lv, pl.ds(e0, 1024)], bufC, sem)
        descs = []
        for j in range(8):
            descs.append(pltpu.async_copy(
                a_tab.at[idx_d.at[j]], bufA.at[pl.ds(j * 128, 128)], sem))
            descs.append(pltpu.async_copy(
                b_tab.at[idx_s.at[j]], bufB.at[pl.ds(j * 128, 128)], sem))
        cpd.wait()
        for dsc in descs:
            dsc.wait()

        @pl.loop(0, 128)
        def _r(rr):
            base = rr * 8
            for u in range(8):
                r = base + u
                v = (bufA[r, pl.ds(0, 32)] + bufB[r, pl.ds(0, 32)]
                     + bufC[r, pl.ds(0, 32)])
                bufP[r, pl.ds(0, 32)] = jnp.maximum(v, jnp.bfloat16(0.0))

        pltpu.sync_copy(bufP, m_out.at[c, pl.ds(e0, 1024)])


_msg = pl.kernel(
    _msg_body,
    out_type=[jax.ShapeDtypeStruct((2, EP, 32), jnp.bfloat16)],
    mesh=_mesh,
    scratch_types=[pltpu.VMEM((8, 128), jnp.int32),
                   pltpu.VMEM((8, 128), jnp.int32),
                   pltpu.VMEM((1024, 32), jnp.bfloat16),
                   pltpu.VMEM((1024, 32), jnp.bfloat16),
                   pltpu.VMEM((1024, 32), jnp.bfloat16),
                   pltpu.VMEM((1024, 32), jnp.bfloat16),
                   pltpu.VMEM((16,), jnp.int32),
                   pltpu.SemaphoreType.DMA],
    compiler_params=pltpu.CompilerParams(use_tc_tiling_on_sc=False, needs_layout_passes=False),
)


NP2 = NP // 2
ACC_R = NP2 + 128
DSL = NP2 // 16


def _agg_body(m, dst2, s_out, idx_d, idxl, bufM, accsp, sem):
    c = lax.axis_index("c")
    s = lax.axis_index("s")
    for half in range(2):
        off = half * NP2

        @pl.loop(0, 800)
        def _z(r):
            bufM[r, pl.ds(0, 32)] = jnp.zeros((32,), jnp.bfloat16)

        for q in range(2):
            pltpu.sync_copy(bufM.at[pl.ds(0, 800)],
                            accsp.at[pl.ds(s * DSL + q * 800, 800)])
        plsc.subcore_barrier()

        @pl.loop(0, NSUP)
        def _g(g):
            row0 = s * SROW + g * 8
            e0 = row0 * 128
            pltpu.sync_copy(dst2.at[pl.ds(row0, 8)], idx_d)
            cpm = pltpu.async_copy(m.at[c, pl.ds(e0, 1024)], bufM, sem)

            @pl.loop(0, 8)
            def _t(j):
                for h in range(8):
                    loc = idx_d[j, pl.ds(h * 16, 16)] - off
                    bad = (loc < 0) | (loc >= NP2)
                    idxl[j, pl.ds(h * 16, 16)] = jnp.where(bad, NP2, loc)

            cpm.wait()
            for j in range(8):
                pltpu.sync_copy(bufM.at[pl.ds(j * 128, 128)],
                                accsp.at[idxl.at[j]], add=True)

        plsc.subcore_barrier()
        pltpu.sync_copy(accsp.at[pl.ds(s * DSL, DSL)],
                        s_out.at[c, pl.ds(off + s * DSL, DSL)])


_agg = pl.kernel(
    _agg_body,
    out_type=[jax.ShapeDtypeStruct((2, NP, 32), jnp.bfloat16)],
    mesh=_mesh,
    scratch_types=[pltpu.VMEM((8, 128), jnp.int32),
                   pltpu.VMEM((8, 128), jnp.int32),
                   pltpu.VMEM((1024, 32), jnp.bfloat16),
                   pltpu.VMEM_SHARED((ACC_R, 32), jnp.bfloat16),
                   pltpu.SemaphoreType.DMA],
    compiler_params=pltpu.CompilerParams(use_tc_tiling_on_sc=False, needs_layout_passes=False),
)


def _lin_in_body(x_ref, w_ref, b_ref, o_ref):
    o_ref[...] = jnp.dot(x_ref[...], w_ref[...],
                         preferred_element_type=_f32) + b_ref[...]


def _ab_body(h_ref, wi_ref, wj_ref, a_ref, b_ref):
    i = pl.program_id(0)
    h = h_ref[...]
    A = jnp.dot(h, wi_ref[...], preferred_element_type=_f32)
    B = jnp.dot(h, wj_ref[...], preferred_element_type=_f32)
    lo = i < NBLK
    a_ref[...] = jnp.where(lo, A[:, :32], A[:, 32:]).astype(jnp.bfloat16)
    b_ref[...] = jnp.where(lo, B[:, :32], B[:, 32:]).astype(jnp.bfloat16)


def _c_body(ea_ref, pi_ref, pj_ref, w_ref, wd_ref, b_ref, c_ref):
    dp = pi_ref[...] - pj_ref[...]
    d2 = jnp.sum(dp * dp, axis=1, keepdims=True)
    R = jnp.dot(ea_ref[...], w_ref[...], preferred_element_type=_f32)
    R = R + d2 * wd_ref[...] + b_ref[...]
    for cc in range(2):
        for l in range(4):
            c_ref[cc, l] = R[:, l * 64 + cc * 32:
                             l * 64 + (cc + 1) * 32].astype(jnp.bfloat16)


def _upd_body(h_ref, s2_ref, deg_ref, w2_ref, b2_ref,
              u1h_ref, u1a_ref, ub1_ref, u2_ref, ub2_ref, o_ref):
    S = jnp.concatenate([s2_ref[0], s2_ref[1]], axis=1).astype(_f32)
    degs = deg_ref[0] + deg_ref[1]
    agg = jnp.dot(S, w2_ref[...], preferred_element_type=_f32) \
        + degs * b2_ref[...]
    t = jnp.maximum(
        jnp.dot(h_ref[...], u1h_ref[...], preferred_element_type=_f32)
        + jnp.dot(agg, u1a_ref[...], preferred_element_type=_f32)
        + ub1_ref[...], 0.0)
    o_ref[...] = jnp.dot(t, u2_ref[...], preferred_element_type=_f32) \
        + ub2_ref[...]


def _pool_body(h_ref, b3_ref, hw_ref, hb_ref, o_ref, acc, cnt):
    i = pl.program_id(0)

    @pl.when(i == 0)
    def _():
        acc[...] = jnp.zeros_like(acc)
        cnt[...] = jnp.zeros_like(cnt)

    bv = b3_ref[0]
    iota = lax.broadcasted_iota(jnp.int32, (1, G), 1)
    oh = (bv == iota).astype(_f32)
    acc[...] += lax.dot_general(oh, h_ref[...], (((0,), (0,)), ((), ())),
                                preferred_element_type=_f32)
    cnt[...] += lax.dot_general(oh, jnp.ones((512, 1), _f32),
                                (((0,), (0,)), ((), ())),
                                preferred_element_type=_f32)

    @pl.when(i == NBLK - 1)
    def _():
        pooled = acc[...] / jnp.maximum(cnt[...], 1.0)
        o_ref[...] = jnp.dot(pooled, hw_ref[...],
                             preferred_element_type=_f32) + hb_ref[...]


_lin_in = pl.pallas_call(
    _lin_in_body,
    grid=(NBLK,),
    in_specs=[pl.BlockSpec((512, 16), lambda i: (i, 0)),
              pl.BlockSpec((16, 64), lambda i: (0, 0)),
              pl.BlockSpec((1, 64), lambda i: (0, 0))],
    out_specs=pl.BlockSpec((512, 64), lambda i: (i, 0)),
    out_shape=jax.ShapeDtypeStruct((NP, 64), _f32),
)

_ab = pl.pallas_call(
    _ab_body,
    grid=(2 * NBLK,),
    in_specs=[pl.BlockSpec((512, 64), lambda i: (i % NBLK, 0)),
              pl.BlockSpec((64, 64), lambda i: (0, 0)),
              pl.BlockSpec((64, 64), lambda i: (0, 0))],
    out_specs=[pl.BlockSpec((512, 32), lambda i: (i, 0)),
               pl.BlockSpec((512, 32), lambda i: (i, 0))],
    out_shape=[jax.ShapeDtypeStruct((2 * NP, 32), jnp.bfloat16),
               jax.ShapeDtypeStruct((2 * NP, 32), jnp.bfloat16)],
)

_cmat = pl.pallas_call(
    _c_body,
    grid=(EBLK,),
    in_specs=[pl.BlockSpec((1024, 16), lambda i: (i, 0)),
              pl.BlockSpec((1024, 16), lambda i: (i, 0)),
              pl.BlockSpec((1024, 16), lambda i: (i, 0)),
              pl.BlockSpec((16, 256), lambda i: (0, 0)),
              pl.BlockSpec((1, 256), lambda i: (0, 0)),
              pl.BlockSpec((1, 256), lambda i: (0, 0))],
    out_specs=pl.BlockSpec((2, 4, 1024, 32), lambda i: (0, 0, i, 0)),
    out_shape=jax.ShapeDtypeStruct((2, 4, EP, 32), jnp.bfloat16),
)

_upd = pl.pallas_call(
    _upd_body,
    grid=(NBLK,),
    in_specs=[pl.BlockSpec((512, 64), lambda i: (i, 0)),
              pl.BlockSpec((2, 512, 32), lambda i: (0, i, 0)),
              pl.BlockSpec((2, 512, 1), lambda i: (0, i, 0)),
              pl.BlockSpec((64, 64), lambda i: (0, 0)),
              pl.BlockSpec((1, 64), lambda i: (0, 0)),
              pl.BlockSpec((64, 64), lambda i: (0, 0)),
              pl.BlockSpec((64, 64), lambda i: (0, 0)),
              pl.BlockSpec((1, 64), lambda i: (0, 0)),
              pl.BlockSpec((64, 64), lambda i: (0, 0)),
              pl.BlockSpec((1, 64), lambda i: (0, 0))],
    out_specs=pl.BlockSpec((512, 64), lambda i: (i, 0)),
    out_shape=jax.ShapeDtypeStruct((NP, 64), _f32),
)

_pool = pl.pallas_call(
    _pool_body,
    grid=(NBLK,),
    in_specs=[pl.BlockSpec((512, 64), lambda i: (i, 0)),
              pl.BlockSpec((1, 512, 1), lambda i: (i, 0, 0)),
              pl.BlockSpec((64, 600), lambda i: (0, 0)),
              pl.BlockSpec((1, 600), lambda i: (0, 0))],
    out_specs=pl.BlockSpec((G, 600), lambda i: (0, 0)),
    out_shape=jax.ShapeDtypeStruct((G, 600), _f32),
    scratch_shapes=[pltpu.VMEM((G, 64), _f32), pltpu.VMEM((G, 1), _f32)],
    compiler_params=pltpu.CompilerParams(
        dimension_semantics=("arbitrary",)),
)


def kernel(x, pos, edge_index, edge_attr, batch, W_in, b_in,
           msg_W1, msg_b1, msg_W2, msg_b2,
           upd_W1, upd_b1, upd_W2, upd_b2,
           head_e_W, head_e_b, head_i_W, head_i_b):
    f32 = _f32
    xp = jnp.pad(x, ((0, NP - N), (0, 5)))
    pos16 = jnp.pad(pos, ((0, NP - N), (0, 13)))
    dst2 = jnp.concatenate(
        [edge_index[1], jnp.full((EP - E,), N, jnp.int32)]).reshape(MROWS, 128)
    src2 = jnp.concatenate(
        [edge_index[0], jnp.full((EP - E,), N, jnp.int32)]).reshape(MROWS, 128)
    eap = jnp.pad(edge_attr, ((0, EP - E), (0, 0)))
    W_inp = jnp.pad(W_in, ((0, 5), (0, 0)))
    W1e_cat = jnp.concatenate([msg_W1[l, 128:144] for l in range(4)], axis=1)
    w1d_cat = jnp.concatenate([msg_W1[l, 144:145] for l in range(4)], axis=1)
    b1_cat = msg_b1.reshape(1, 256)
    head_W = jnp.concatenate([head_e_W, head_i_W], axis=1)
    head_b = jnp.concatenate([head_e_b, head_i_b]).reshape(1, 600)
    batch3 = jnp.pad(batch, (0, NP - N),
                     constant_values=G).reshape(NBLK, 512, 1)

    pos_i, pos_j, deg2 = _prep(pos16, dst2, src2)
    deg3 = deg2.reshape(2, NP, 1)

    c_all = _cmat(eap, pos_i, pos_j, W1e_cat, w1d_cat, b1_cat)

    h = _lin_in(xp, W_inp, b_in.reshape(1, 64))
    lsel_all = jnp.concatenate(
        [jnp.arange(4, dtype=jnp.int32).reshape(4, 1),
         jnp.zeros((4, 15), jnp.int32)], axis=1)
    xs = (msg_W1[:, :64], msg_W1[:, 64:128], msg_W2,
          msg_b2.reshape(4, 1, 64), upd_W1[:, :64], upd_W1[:, 64:],
          upd_b1.reshape(4, 1, 64), upd_W2, upd_b2.reshape(4, 1, 64),
          lsel_all)

    def _layer(hc, x):
        w1i, w1j, w2p, b2, u1h, u1a, ub1, u2, ub2, lsel = x
        a_tab, b_tab = _ab(hc, w1i, w1j)
        (m,) = _msg(a_tab, b_tab, c_all, dst2, src2, lsel)
        (s2,) = _agg(m, dst2)
        hn = _upd(hc, s2, deg3, w2p, b2, u1h, u1a, ub1, u2, ub2)
        return hn, None

    h, _ = lax.scan(_layer, h, xs)

    return _pool(h, batch3, head_W, head_b)
